# baseline (speedup 1.0000x reference)
import functools

import jax
import jax.numpy as jnp
from jax import lax
from jax.experimental import pallas as pl
from jax.experimental.pallas import tpu as pltpu
from jax.experimental.pallas import tpu_sc as plsc

_EMBED_DIM = 5
_N_CAT = 47
_BATCH = 16384

_NC = 2
_NS = 16
_NW = _NC * _NS
_B_PER_W = _BATCH // _NW
_D_PAD = 48
_L = 16
_NBLK = _B_PER_W // _L

_sc_mesh = plsc.VectorSubcoreMesh(core_axis_name="c", subcore_axis_name="s")


@functools.partial(
    pl.kernel,
    out_type=jax.ShapeDtypeStruct((_BATCH, 1, _N_CAT), jnp.float32),
    mesh=_sc_mesh,
    scratch_types=[
        pltpu.VMEM((_B_PER_W,), jnp.int32),
        pltpu.VMEM((_N_CAT * _EMBED_DIM + _L,), jnp.float32),
        pltpu.VMEM((16 * _L,), jnp.float32),
        pltpu.VMEM((_D_PAD,), jnp.float32),
        pltpu.VMEM((_N_CAT * _D_PAD,), jnp.float32),
        pltpu.VMEM((_B_PER_W, 1, _N_CAT), jnp.float32),
        pltpu.SemaphoreType.DMA,
    ],
    compiler_params=pltpu.CompilerParams(
        use_tc_tiling_on_sc=True, needs_layout_passes=False
    ),
)
def _sc_fused_lookup(
    idx_hbm, emb_hbm, w_hbm, b_hbm, out_hbm,
    idx_v, emb_v, w_v, bias_v, table_v, out_v, sem,
):
    wid = lax.axis_index("s") * _NC + lax.axis_index("c")

    idx_cp = pltpu.async_copy(
        idx_hbm.at[pl.ds(wid * _B_PER_W, _B_PER_W)], idx_v, sem
    )
    pltpu.sync_copy(emb_hbm, emb_v.at[pl.ds(0, _N_CAT * _EMBED_DIM)])
    pltpu.sync_copy(w_hbm, w_v.at[pl.ds(0, _EMBED_DIM * _N_CAT)])
    pltpu.sync_copy(b_hbm, bias_v.at[pl.ds(0, _N_CAT)])

    w_vregs = [
        [w_v[pl.ds(e * _N_CAT + k * _L, _L)] for k in range(3)]
        for e in range(_EMBED_DIM)
    ]
    b_vregs = [bias_v[pl.ds(k * _L, _L)] for k in range(3)]

    def table_row(r, _):
        accs = list(b_vregs)
        erow = emb_v[pl.ds(r * _EMBED_DIM, _L)]
        for e in range(_EMBED_DIM):
            s = erow[e]
            for k in range(3):
                accs[k] = accs[k] + s * w_vregs[e][k]
        for k in range(3):
            table_v[pl.ds(r * _D_PAD + k * _L, _L)] = accs[k]
        return _

    lax.fori_loop(0, _N_CAT, table_row, 0)
    idx_cp.wait()

    iota = lax.iota(jnp.int32, _L)

    zeros = iota * 0

    def gather_block(i, _):
        b0 = i * _L
        idx16 = idx_v[pl.ds(b0, _L)]
        src = idx16 * _D_PAD
        rows = b0 + iota
        col = zeros
        for j in range(_N_CAT):
            vals = plsc.load_gather(table_v, [src])
            plsc.store_scatter(out_v, [rows, zeros, col], vals)
            if j != _N_CAT - 1:
                src = src + 1
                col = col + 1
        return _

    lax.fori_loop(0, _NBLK, gather_block, 0)

    pltpu.sync_copy(
        out_v,
        out_hbm.at[pl.ds(wid * _B_PER_W, _B_PER_W)],
    )


def kernel(inputs, emb_table, dense_kernel, dense_bias):
    out = _sc_fused_lookup(
        inputs.reshape(_BATCH),
        emb_table.reshape(_N_CAT * _EMBED_DIM),
        dense_kernel.reshape(_EMBED_DIM * _N_CAT),
        dense_bias,
    )
    return out


# speedup vs baseline: 1.8218x; 1.8218x over previous
import functools

import jax
import jax.numpy as jnp
from jax import lax
from jax.experimental import pallas as pl
from jax.experimental.pallas import tpu as pltpu
from jax.experimental.pallas import tpu_sc as plsc

_EMBED_DIM = 5
_N_CAT = 47
_BATCH = 16384

_NC = 2
_NS = 16
_NW = _NC * _NS
_B_PER_W = _BATCH // _NW
_D_PAD = 48
_L = 16
_NBLK = _B_PER_W // _L

_sc_mesh = plsc.VectorSubcoreMesh(core_axis_name="c", subcore_axis_name="s")


@functools.partial(
    pl.kernel,
    out_type=jax.ShapeDtypeStruct((_N_CAT, 1, _BATCH), jnp.float32),
    mesh=_sc_mesh,
    scratch_types=[
        pltpu.VMEM((_B_PER_W,), jnp.int32),
        pltpu.VMEM((_N_CAT * _EMBED_DIM + _L,), jnp.float32),
        pltpu.VMEM((16 * _L,), jnp.float32),
        pltpu.VMEM((_D_PAD,), jnp.float32),
        pltpu.VMEM((_N_CAT * _D_PAD,), jnp.float32),
        pltpu.VMEM((_N_CAT * _B_PER_W,), jnp.float32),
        pltpu.SemaphoreType.DMA,
    ],
    compiler_params=pltpu.CompilerParams(
        use_tc_tiling_on_sc=False, needs_layout_passes=False
    ),
)
def _sc_fused_lookup(
    idx_hbm, emb_hbm, w_hbm, b_hbm, out_hbm,
    idx_v, emb_v, w_v, bias_v, table_v, out_v, sem,
):
    wid = lax.axis_index("s") * _NC + lax.axis_index("c")

    idx_cp = pltpu.async_copy(
        idx_hbm.at[pl.ds(wid * _B_PER_W, _B_PER_W)], idx_v, sem
    )
    pltpu.sync_copy(emb_hbm, emb_v.at[pl.ds(0, _N_CAT * _EMBED_DIM)])
    pltpu.sync_copy(w_hbm, w_v.at[pl.ds(0, _EMBED_DIM * _N_CAT)])
    pltpu.sync_copy(b_hbm, bias_v.at[pl.ds(0, _N_CAT)])

    w_vregs = [
        [w_v[pl.ds(e * _N_CAT + k * _L, _L)] for k in range(3)]
        for e in range(_EMBED_DIM)
    ]
    b_vregs = [bias_v[pl.ds(k * _L, _L)] for k in range(3)]

    def table_row(r, _):
        accs = list(b_vregs)
        erow = emb_v[pl.ds(r * _EMBED_DIM, _L)]
        for e in range(_EMBED_DIM):
            s = erow[e]
            for k in range(3):
                accs[k] = accs[k] + s * w_vregs[e][k]
        for k in range(3):
            table_v[pl.ds(r * _D_PAD + k * _L, _L)] = accs[k]
        return _

    lax.fori_loop(0, _N_CAT, table_row, 0)
    idx_cp.wait()

    def gather_block(i, _):
        b0 = i * _L
        idx16 = idx_v[pl.ds(b0, _L)]
        src = idx16 * _D_PAD
        for j in range(_N_CAT):
            vals = plsc.load_gather(table_v, [src])
            out_v[pl.ds(j * _B_PER_W + b0, _L)] = vals
            if j != _N_CAT - 1:
                src = src + 1
        return _

    lax.fori_loop(0, _NBLK, gather_block, 0)

    copies = [
        pltpu.async_copy(
            out_v.at[pl.ds(j * _B_PER_W, _B_PER_W)],
            out_hbm.at[j, 0, pl.ds(wid * _B_PER_W, _B_PER_W)],
            sem,
        )
        for j in range(_N_CAT)
    ]
    for cp in copies:
        cp.wait()


def kernel(inputs, emb_table, dense_kernel, dense_bias):
    out = _sc_fused_lookup(
        inputs.reshape(_BATCH),
        emb_table.reshape(_N_CAT * _EMBED_DIM),
        dense_kernel.reshape(_EMBED_DIM * _N_CAT),
        dense_bias,
    )
    return jnp.transpose(out, (2, 1, 0))


# speedup vs baseline: 2.1752x; 1.1940x over previous
import functools

import jax
import jax.numpy as jnp
from jax import lax
from jax.experimental import pallas as pl
from jax.experimental.pallas import tpu as pltpu
from jax.experimental.pallas import tpu_sc as plsc

_EMBED_DIM = 5
_N_CAT = 47
_BATCH = 16384

_NC = 2
_NS = 16
_NW = _NC * _NS
_B_PER_W = _BATCH // _NW
_D_PAD = 48
_L = 16
_NBLK = _B_PER_W // _L

_EMB_OFF = 0
_W_OFF = 240
_B_OFF = 480
_PACK_LEN = _B_OFF + _N_CAT

_sc_mesh = plsc.VectorSubcoreMesh(core_axis_name="c", subcore_axis_name="s")


@functools.partial(
    pl.kernel,
    out_type=jax.ShapeDtypeStruct((_N_CAT, 1, _BATCH), jnp.float32),
    mesh=_sc_mesh,
    scratch_types=[
        pltpu.VMEM((_B_PER_W,), jnp.int32),
        pltpu.VMEM((_PACK_LEN + _L,), jnp.float32),
        pltpu.VMEM((_N_CAT * _D_PAD,), jnp.float32),
        pltpu.VMEM((_N_CAT * _B_PER_W,), jnp.float32),
        pltpu.SemaphoreType.DMA,
        pltpu.SemaphoreType.DMA,
    ],
    compiler_params=pltpu.CompilerParams(
        use_tc_tiling_on_sc=False, needs_layout_passes=False
    ),
)
def _sc_fused_lookup(
    idx_hbm, pack_hbm, out_hbm, idx_v, wb_v, table_v, out_v, sem_in, sem_out
):
    wid = lax.axis_index("s") * _NC + lax.axis_index("c")

    idx_cp = pltpu.async_copy(
        idx_hbm.at[pl.ds(wid * _B_PER_W, _B_PER_W)], idx_v, sem_in
    )
    pltpu.sync_copy(pack_hbm, wb_v.at[pl.ds(0, _PACK_LEN)])

    w_vregs = [
        [wb_v[pl.ds(_W_OFF + e * _N_CAT + k * _L, _L)] for k in range(3)]
        for e in range(_EMBED_DIM)
    ]
    b_vregs = [wb_v[pl.ds(_B_OFF + k * _L, _L)] for k in range(3)]

    def table_row(r, _):
        accs = list(b_vregs)
        erow = wb_v[pl.ds(_EMB_OFF + r * _EMBED_DIM, _L)]
        for e in range(_EMBED_DIM):
            s = erow[e]
            for k in range(3):
                accs[k] = accs[k] + s * w_vregs[e][k]
        for k in range(3):
            table_v[pl.ds(r * _D_PAD + k * _L, _L)] = accs[k]
        return _

    lax.fori_loop(0, _N_CAT, table_row, 0)
    idx_cp.wait()

    srcs0 = tuple(
        idx_v[pl.ds(i * _L, _L)] * _D_PAD for i in range(_NBLK)
    )
    out_base = wid * _B_PER_W

    def column(j, srcs):
        col_off = j * _B_PER_W
        for i0 in range(0, _NBLK, 4):
            vals = [
                plsc.load_gather(table_v, [srcs[i0 + k]]) for k in range(4)
            ]
            for k in range(4):
                out_v[pl.ds(col_off + (i0 + k) * _L, _L)] = vals[k]
        pltpu.async_copy(
            out_v.at[pl.ds(col_off, _B_PER_W)],
            out_hbm.at[j, 0, pl.ds(out_base, _B_PER_W)],
            sem_out,
        )
        return tuple(s + 1 for s in srcs)

    lax.fori_loop(0, _N_CAT, column, srcs0)

    for j in range(_N_CAT):
        pltpu.make_async_copy(
            out_hbm.at[0, 0, pl.ds(0, _B_PER_W)],
            out_v.at[pl.ds(j * _B_PER_W, _B_PER_W)],
            sem_out,
        ).wait()


def kernel(inputs, emb_table, dense_kernel, dense_bias):
    pack = jnp.concatenate(
        [
            jnp.pad(emb_table.reshape(_N_CAT * _EMBED_DIM), (0, 5)),
            jnp.pad(dense_kernel.reshape(_EMBED_DIM * _N_CAT), (0, 5)),
            dense_bias,
        ]
    )
    out = _sc_fused_lookup(inputs.reshape(_BATCH), pack)
    return jnp.transpose(out, (2, 1, 0))
